# baseline (speedup 1.0000x reference)
import jax
import jax.numpy as jnp
from jax.experimental import pallas as pl


def _finalize_body(hg_ref, w1_ref, b1_ref, gamma_ref, beta_ref, w2_ref, b2_ref,
                   out_ref, sig_ref):
    z = hg_ref[...] @ w1_ref[...] + b1_ref[...]
    mu = jnp.mean(z, axis=0)
    var = jnp.var(z, axis=0)
    z = (z - mu) * jax.lax.rsqrt(var + 1e-5) * gamma_ref[...] + beta_ref[...]
    z = jnp.where(z > 0, z, 0.01 * z)
    out = z @ w2_ref[...] + b2_ref[...]
    out_ref[...] = out
    sig_ref[...] = jax.nn.sigmoid(out)


def kernel(x, edge_index, batch, edge_attr, params):
    relu = jax.nn.relu
    p = params
    h = relu(x @ p['node_W1'] + p['node_b1'])
    h = relu(h @ p['node_W2'] + p['node_b2'])
    h = h @ p['node_W3'] + p['node_b3']
    e = relu(edge_attr @ p['edge_W1'] + p['edge_b1'])
    e = relu(e @ p['edge_W2'] + p['edge_b2'])
    e = e @ p['edge_W3'] + p['edge_b3']
    e = e @ p['gat_We']
    src = edge_index[0]
    dst = edge_index[1]
    xl = h @ p['gat_Wl'] + p['gat_bl']
    xr = h @ p['gat_Wr']
    m = xl[src] + xr[dst] + e
    m_act = jax.nn.leaky_relu(m, negative_slope=0.2)
    logits = m_act @ p['gat_att']
    seg_max = jax.ops.segment_max(logits, dst, num_segments=10000)
    seg_max = jnp.where(jnp.isfinite(seg_max), seg_max, 0.0)
    unnorm = jnp.exp(logits - seg_max[dst])
    denom = jax.ops.segment_sum(unnorm, dst, num_segments=10000)
    alpha = unnorm / (denom[dst] + 1e-16)
    node_out = jax.ops.segment_sum(alpha[:, None] * xl[src], dst,
                                   num_segments=10000) + p['gat_bias']
    hg = jax.ops.segment_max(node_out, batch, num_segments=64)

    out, sig = pl.pallas_call(
        _finalize_body,
        out_shape=(jax.ShapeDtypeStruct((64, 1), jnp.float32),
                   jax.ShapeDtypeStruct((64, 1), jnp.float32)),
    )(hg, p['out_W1'], p['out_b1'], p['bn_gamma'], p['bn_beta'],
      p['out_W2'], p['out_b2'])
    return (out, sig)


# speedup vs baseline: 26.4367x; 26.4367x over previous
import jax
import jax.numpy as jnp
from jax import lax
from jax.experimental import pallas as pl
from jax.experimental.pallas import tpu as pltpu
from jax.experimental.pallas import tpu_sc as plsc

N = 10000
E = 640000
G = 64
H = 16
EPAD = 655360
RPAD = EPAD // 128
VALID_ROWS = E // 128
ROWS_PER_TILE = RPAD // 32
CHUNK_ROWS = 4
KCH = CHUNK_ROWS * 128
N_CHUNKS = ROWS_PER_TILE // CHUNK_ROWS
EB = 5120
NSP = 10240
NROWS_PER_TILE = NSP // 16


def _node_prep_body(x_ref, w1, b1, w2, b2, w3, b3, wl, bl, wr,
                    xl_ref, xr_ref):
    h = jnp.maximum(x_ref[...] @ w1[...] + b1[...], 0.0)
    h = jnp.maximum(h @ w2[...] + b2[...], 0.0)
    h = h @ w3[...] + b3[...]
    xl_ref[...] = h @ wl[...] + bl[...]
    xr_ref[...] = h @ wr[...]


def _node_prep(x, p):
    return pl.pallas_call(
        _node_prep_body,
        out_shape=(jax.ShapeDtypeStruct((N, H), jnp.float32),
                   jax.ShapeDtypeStruct((N, H), jnp.float32)),
    )(x, p['node_W1'], p['node_b1'], p['node_W2'], p['node_b2'],
      p['node_W3'], p['node_b3'], p['gat_Wl'], p['gat_bl'], p['gat_Wr'])


def _edge_mlp_body(aT_ref, w1t, b1c, w2t, b2c, w3t, b3c, wet, eT_ref):
    h = jnp.maximum(w1t[...] @ aT_ref[...] + b1c[...], 0.0)
    h = jnp.maximum(w2t[...] @ h + b2c[...], 0.0)
    h = w3t[...] @ h + b3c[...]
    eT_ref[...] = wet[...] @ h


def _edge_mlp(aT_pad, p):
    grid = EPAD // EB
    full = pl.BlockSpec((H, H), lambda i: (0, 0))
    colv = pl.BlockSpec((H, 1), lambda i: (0, 0))
    return pl.pallas_call(
        _edge_mlp_body,
        grid=(grid,),
        in_specs=[pl.BlockSpec((H, EB), lambda i: (0, i)),
                  full, colv, full, colv, full, colv, full],
        out_specs=pl.BlockSpec((H, EB), lambda i: (0, i)),
        out_shape=jax.ShapeDtypeStruct((H, EPAD), jnp.float32),
    )(aT_pad,
      p['edge_W1'].T, p['edge_b1'][:, None],
      p['edge_W2'].T, p['edge_b2'][:, None],
      p['edge_W3'].T, p['edge_b3'][:, None],
      p['gat_We'].T)


def _edge_pass_body(srcR, dstR, e_hbm, xl_hbm, xr_hbm, att_hbm, acc_out,
                    src_v, dst_v0, dst_v1, dst_v2, dst_v3,
                    e_v, xlr, xrr, wr, att_v, zb, acc_sp,
                    sem_in, sem_g, sem_sc):
    c = lax.axis_index("c")
    s = lax.axis_index("s")
    wid = s * 2 + c
    dst_vs = (dst_v0, dst_v1, dst_v2, dst_v3)

    zero16 = jnp.zeros((16,), jnp.float32)

    def _zero_zb(r, carry):
        zb[r, pl.ds(0, 16)] = zero16
        zb[r, pl.ds(16, 16)] = zero16
        return carry
    lax.fori_loop(0, NROWS_PER_TILE, _zero_zb, None)

    pltpu.sync_copy(zb, acc_sp.at[pl.ds(s * NROWS_PER_TILE, NROWS_PER_TILE)])
    pltpu.sync_copy(att_hbm, att_v)
    plsc.subcore_barrier()

    lane = lax.iota(jnp.int32, 16)

    def chunk_body(ci, carry):
        row0 = wid * ROWS_PER_TILE + ci * CHUNK_ROWS
        cps = [pltpu.async_copy(srcR.at[pl.ds(row0, CHUNK_ROWS)], src_v,
                                sem_in),
               pltpu.async_copy(e_hbm.at[:, pl.ds(row0 * 128, KCH)], e_v,
                                sem_in)]
        for j in range(CHUNK_ROWS):
            cps.append(pltpu.async_copy(dstR.at[row0 + j], dst_vs[j], sem_in))
        for cp in cps:
            cp.wait()
        gps = []
        for j in range(CHUNK_ROWS):
            gps.append(pltpu.async_copy(
                xl_hbm.at[src_v.at[j]], xlr.at[pl.ds(j * 128, 128)], sem_g))
            gps.append(pltpu.async_copy(
                xr_hbm.at[dst_vs[j]], xrr.at[pl.ds(j * 128, 128)], sem_g))
        for cp in gps:
            cp.wait()

        def group_body(g, carry2):
            base = g * 16
            rowi = base + lane
            valid = ((row0 + g // 8) < VALID_ROWS).astype(jnp.float32)
            logit = jnp.zeros((16,), jnp.float32)
            xls = []
            for j in range(H):
                colj = jnp.full((16,), j, jnp.int32)
                xl_j = plsc.load_gather(xlr, [rowi, colj])
                xr_j = plsc.load_gather(xrr, [rowi, colj])
                e_j = e_v[j, pl.ds(base, 16)]
                m = xl_j + xr_j + e_j
                act = jnp.where(m >= 0, m, 0.2 * m)
                ub = lax.bitcast_convert_type(act, jnp.uint32)
                bias = jnp.uint32(0x7FFF) + ((ub >> jnp.uint32(16))
                                             & jnp.uint32(1))
                ub = (ub + bias) & jnp.uint32(0xFFFF0000)
                actq = lax.bitcast_convert_type(ub, jnp.float32)
                logit = logit + att_v[j, :] * actq
                xls.append(xl_j)
            u = jnp.exp(logit) * valid
            for j in range(H):
                plsc.store_scatter(wr, [rowi, jnp.full((16,), j, jnp.int32)],
                                   u * xls[j])
            plsc.store_scatter(wr, [rowi, jnp.full((16,), 16, jnp.int32)], u)
            return carry2
        lax.fori_loop(0, KCH // 16, group_body, None)

        sps = []
        for j in range(CHUNK_ROWS):
            sps.append(pltpu.async_copy(
                wr.at[pl.ds(j * 128, 128)], acc_sp.at[dst_vs[j]],
                sem_sc, add=True))
        for cp in sps:
            cp.wait()
        return carry

    lax.fori_loop(0, N_CHUNKS, chunk_body, None)

    plsc.subcore_barrier()
    sl = pl.ds(s * NROWS_PER_TILE, NROWS_PER_TILE)
    pltpu.sync_copy(acc_sp.at[sl], acc_out.at[c, sl])


def _edge_pass(srcR, dstR, eT_pad, xl, xr, attT):
    mesh = plsc.VectorSubcoreMesh(core_axis_name="c", subcore_axis_name="s")
    f = pl.kernel(
        _edge_pass_body,
        out_type=jax.ShapeDtypeStruct((2, NSP, 32), jnp.float32),
        mesh=mesh,
        compiler_params=pltpu.CompilerParams(needs_layout_passes=False,
                                             use_tc_tiling_on_sc=False),
        scratch_types=[
            pltpu.VMEM((CHUNK_ROWS, 128), jnp.int32),
            pltpu.VMEM((128,), jnp.int32),
            pltpu.VMEM((128,), jnp.int32),
            pltpu.VMEM((128,), jnp.int32),
            pltpu.VMEM((128,), jnp.int32),
            pltpu.VMEM((H, KCH), jnp.float32),
            pltpu.VMEM((KCH, H), jnp.float32),
            pltpu.VMEM((KCH, H), jnp.float32),
            pltpu.VMEM((KCH, 32), jnp.float32),
            pltpu.VMEM((H, H), jnp.float32),
            pltpu.VMEM((NROWS_PER_TILE, 32), jnp.float32),
            pltpu.VMEM_SHARED((NSP, 32), jnp.float32),
            pltpu.SemaphoreType.DMA,
            pltpu.SemaphoreType.DMA,
            pltpu.SemaphoreType.DMA,
        ],
    )
    return f(srcR, dstR, eT_pad, xl, xr, attT)


def _finalize_body(acc_ref, batch_ref, gbias, w1, b1, gamma, beta, w2, b2,
                   out_ref, sig_ref, hg):
    acc = acc_ref[0, :N, :] + acc_ref[1, :N, :]
    node = acc[:, :16] / (acc[:, 16:17] + 1e-16) + gbias[...]
    bcol = batch_ref[...]

    def pool(g, carry):
        m = jnp.where(bcol == g, node, -jnp.inf)
        hg[pl.ds(g, 1), :] = jnp.max(m, axis=0, keepdims=True)
        return carry
    lax.fori_loop(0, G, pool, None)

    def q(a):
        return a.astype(jnp.bfloat16).astype(jnp.float32)

    z = q(hg[...]) @ q(w1[...]) + b1[...]
    mu = jnp.mean(z, axis=0)
    var = jnp.mean((z - mu) * (z - mu), axis=0)
    z = (z - mu) * lax.rsqrt(var + 1e-5) * gamma[...] + beta[...]
    z = jnp.where(z >= 0, z, 0.01 * z)
    out = q(z) @ q(w2[...]) + b2[...]
    out_ref[...] = out
    sig_ref[...] = jax.nn.sigmoid(out)


def _finalize(acc, batch_col, p):
    return pl.pallas_call(
        _finalize_body,
        out_shape=(jax.ShapeDtypeStruct((G, 1), jnp.float32),
                   jax.ShapeDtypeStruct((G, 1), jnp.float32)),
        scratch_shapes=[pltpu.VMEM((G, H), jnp.float32)],
    )(acc, batch_col, p['gat_bias'], p['out_W1'], p['out_b1'],
      p['bn_gamma'], p['bn_beta'], p['out_W2'], p['out_b2'])


def kernel(x, edge_index, batch, edge_attr, params):
    p = params
    xl, xr = _node_prep(x, p)

    aT_pad = jnp.zeros((H, EPAD), jnp.float32).at[:, :E].set(edge_attr.T)
    eT_pad = _edge_mlp(aT_pad, p)

    srcR = jnp.zeros((EPAD,), jnp.int32).at[:E].set(
        edge_index[0]).reshape(RPAD, 128)
    dstR = jnp.zeros((EPAD,), jnp.int32).at[:E].set(
        edge_index[1]).reshape(RPAD, 128)

    ua = lax.bitcast_convert_type(p['gat_att'], jnp.uint32)
    ua = (ua + (jnp.uint32(0x7FFF) + ((ua >> jnp.uint32(16)) & jnp.uint32(1)))) \
        & jnp.uint32(0xFFFF0000)
    att_q = lax.bitcast_convert_type(ua, jnp.float32)
    attT = jnp.tile(att_q[:, None], (1, H))
    acc = _edge_pass(srcR, dstR, eT_pad, xl, xr, attT)

    out, sig = _finalize(acc, batch[:, None], p)
    return (out, sig)
